# Initial kernel scaffold; baseline (speedup 1.0000x reference)
import jax
import jax.numpy as jnp
from jax.experimental import pallas as pl


def kernel(node_feats_0, invariant_edge_feats, edge_index, W1, b1, W2, b2, W3):
    raise NotImplementedError("write your pallas kernel here")


# speedup vs baseline: 3.4548x; 3.4548x over previous
import functools

import jax
import jax.numpy as jnp
from jax import lax
from jax.experimental import pallas as pl
from jax.experimental.pallas import tpu as pltpu
from jax.experimental.pallas import tpu_sc as plsc

N = 50000
E = 800000
CIN = 16
COUT = 16
MID = 32

NC = 2
NS = 16
NW = NC * NS

CHUNK = 128
G = 14
GITERS = 14
CPT = G * GITERS
EPT = CPT * CHUNK
E_PAD = NW * EPT

NACC = 50048
RPT = NACC // NS

@functools.cache
def _mesh():
    return plsc.VectorSubcoreMesh(
        core_axis_name="c", subcore_axis_name="s", num_cores=NC, num_subcores=NS
    )


@functools.cache
def _sc_gather_kernel():
    @functools.partial(
        pl.kernel,
        out_type=jax.ShapeDtypeStruct((E_PAD, CIN), jnp.float32),
        mesh=_mesh(),
        scratch_types=[
            pltpu.VMEM((CPT, CHUNK), jnp.int32),
            pltpu.VMEM((G * CHUNK, CIN), jnp.float32),
            pltpu.SemaphoreType.DMA,
        ],
        compiler_params=pltpu.CompilerParams(use_tc_tiling_on_sc=False),
    )
    def _sc_gather(nf_hbm, src_hbm, out_hbm, idx_v, rows_v, sem):
        wid = lax.axis_index("s") * NC + lax.axis_index("c")
        pltpu.sync_copy(src_hbm.at[wid], idx_v)

        def body(g, carry):
            descs = [
                pltpu.async_copy(
                    nf_hbm.at[idx_v.at[g * G + b]],
                    rows_v.at[pl.ds(b * CHUNK, CHUNK)],
                    sem,
                )
                for b in range(G)
            ]
            for d in descs:
                d.wait()
            pltpu.sync_copy(
                rows_v, out_hbm.at[pl.ds(wid * EPT + g * (G * CHUNK), G * CHUNK)]
            )
            return carry

        lax.fori_loop(0, GITERS, body, 0)

    return _sc_gather


_BLK = 2048


def _tc_compute_body(ef_ref, f_ref, w1_ref, w2_ref, b2_ref, w3_ref, out_ref):
    h1 = jnp.maximum(
        jnp.dot(ef_ref[...], w1_ref[...], preferred_element_type=jnp.float32), 0.0
    )
    h2 = jnp.maximum(
        jnp.dot(h1, w2_ref[...], preferred_element_type=jnp.float32)
        + b2_ref[0:1, :],
        0.0,
    )
    rw = jnp.dot(h2, w3_ref[...], preferred_element_type=jnp.float32)
    f = f_ref[...]
    frep = jnp.concatenate([f] * COUT, axis=1)
    prod = rw * frep
    r = lax.broadcasted_iota(jnp.int32, (COUT * CIN, COUT), 0)
    c = lax.broadcasted_iota(jnp.int32, (COUT * CIN, COUT), 1)
    sel = jnp.where((r // CIN) == c, 1.0, 0.0)
    out_ref[...] = jnp.dot(prod, sel, preferred_element_type=jnp.float32)


def _tc_compute(ef8, feats, w18, w2, b2_8, w3):
    grid = E_PAD // _BLK
    return pl.pallas_call(
        _tc_compute_body,
        grid=(grid,),
        in_specs=[
            pl.BlockSpec((_BLK, 8), lambda i: (i, 0)),
            pl.BlockSpec((_BLK, CIN), lambda i: (i, 0)),
            pl.BlockSpec((8, MID), lambda i: (0, 0)),
            pl.BlockSpec((MID, MID), lambda i: (0, 0)),
            pl.BlockSpec((8, MID), lambda i: (0, 0)),
            pl.BlockSpec((MID, COUT * CIN), lambda i: (0, 0)),
        ],
        out_specs=pl.BlockSpec((_BLK, COUT), lambda i: (i, 0)),
        out_shape=jax.ShapeDtypeStruct((E_PAD, COUT), jnp.float32),
    )(ef8, feats, w18, w2, b2_8, w3)


@functools.cache
def _sc_scatter_kernel():
    @functools.partial(
        pl.kernel,
        out_type=jax.ShapeDtypeStruct((NC, NACC, COUT), jnp.float32),
        mesh=_mesh(),
        scratch_types=[
            pltpu.VMEM((CPT, CHUNK), jnp.int32),
            pltpu.VMEM((G * CHUNK, COUT), jnp.float32),
            pltpu.VMEM_SHARED((NACC, COUT), jnp.float32),
        ],
        compiler_params=pltpu.CompilerParams(use_tc_tiling_on_sc=False),
    )
    def _sc_scatter(tmp_hbm, dst_hbm, zero_hbm, out_hbm, idx_v, rows_v, acc):
        cid = lax.axis_index("c")
        sid = lax.axis_index("s")
        wid = sid * NC + cid
        pltpu.sync_copy(dst_hbm.at[wid], idx_v)

        @pl.when(sid == 0)
        def _():
            pltpu.sync_copy(zero_hbm, acc)

        plsc.subcore_barrier()

        def body(g, carry):
            pltpu.sync_copy(
                tmp_hbm.at[pl.ds(wid * EPT + g * (G * CHUNK), G * CHUNK)], rows_v
            )
            for b in range(G):
                pltpu.sync_copy(
                    rows_v.at[pl.ds(b * CHUNK, CHUNK)],
                    acc.at[idx_v.at[g * G + b]],
                    add=True,
                )
            return carry

        lax.fori_loop(0, GITERS, body, 0)
        plsc.subcore_barrier()
        pltpu.sync_copy(
            acc.at[pl.ds(sid * RPT, RPT)],
            out_hbm.at[cid, pl.ds(sid * RPT, RPT)],
        )

    return _sc_scatter


_ABLK = 2000


def _tc_add_body(a_ref, b_ref, out_ref):
    out_ref[...] = a_ref[...] + b_ref[...]


def _tc_add(a, b):
    grid = N // _ABLK
    return pl.pallas_call(
        _tc_add_body,
        grid=(grid,),
        in_specs=[
            pl.BlockSpec((_ABLK, COUT), lambda i: (i, 0)),
            pl.BlockSpec((_ABLK, COUT), lambda i: (i, 0)),
        ],
        out_specs=pl.BlockSpec((_ABLK, COUT), lambda i: (i, 0)),
        out_shape=jax.ShapeDtypeStruct((N, COUT), jnp.float32),
    )(a, b)


def kernel(node_feats_0, invariant_edge_feats, edge_index, W1, b1, W2, b2, W3):
    nf = node_feats_0.reshape(N, CIN)
    src = edge_index[0]
    dst = edge_index[1]
    pad = E_PAD - E

    src_p = jnp.concatenate([src, jnp.zeros((pad,), jnp.int32)]).reshape(NW, CPT, CHUNK)
    dst_p = jnp.concatenate([dst, jnp.full((pad,), N, jnp.int32)]).reshape(NW, CPT, CHUNK)

    ones = jnp.ones((E, 1), jnp.float32)
    zer3 = jnp.zeros((E, 3), jnp.float32)
    ef8 = jnp.concatenate([invariant_edge_feats, ones, zer3], axis=1)
    ef8 = jnp.concatenate([ef8, jnp.zeros((pad, 8), jnp.float32)], axis=0)

    w18 = jnp.concatenate([W1, b1[None, :], jnp.zeros((3, MID), jnp.float32)], axis=0)
    b2_8 = jnp.broadcast_to(b2[None, :], (8, MID))

    feats = _sc_gather_kernel()(nf, src_p)
    tmp = _tc_compute(ef8, feats, w18, W2, b2_8, W3)
    zero = jnp.zeros((NACC, COUT), jnp.float32)
    partials = _sc_scatter_kernel()(tmp, dst_p, zero)
    out = _tc_add(partials[0, :N], partials[1, :N])
    return out.reshape(N, CIN, 1)


# speedup vs baseline: 4.8715x; 1.4100x over previous
import functools

import jax
import jax.numpy as jnp
from jax import lax
from jax.experimental import pallas as pl
from jax.experimental.pallas import tpu as pltpu
from jax.experimental.pallas import tpu_sc as plsc

N = 50000
E = 800000
CIN = 16
COUT = 16
MID = 32

NC = 2
NS = 16
NW = NC * NS

CHUNK = 128
G = 14
GITERS = 14
CPT = G * GITERS
EPT = CPT * CHUNK
E_PAD = NW * EPT

NACC = 50048
RPT = NACC // NS

@functools.cache
def _mesh():
    return plsc.VectorSubcoreMesh(
        core_axis_name="c", subcore_axis_name="s", num_cores=NC, num_subcores=NS
    )


@functools.cache
def _sc_gather_kernel():
    @functools.partial(
        pl.kernel,
        out_type=jax.ShapeDtypeStruct((E_PAD, CIN), jnp.float32),
        mesh=_mesh(),
        scratch_types=[
            pltpu.VMEM((CPT, CHUNK), jnp.int32),
            pltpu.VMEM((G * CHUNK, CIN), jnp.float32),
            pltpu.SemaphoreType.DMA,
        ],
        compiler_params=pltpu.CompilerParams(use_tc_tiling_on_sc=False),
    )
    def _sc_gather(nf_hbm, src_hbm, out_hbm, idx_v, rows_v, sem):
        wid = lax.axis_index("s") * NC + lax.axis_index("c")
        pltpu.sync_copy(src_hbm.at[wid], idx_v)

        def body(g, carry):
            descs = [
                pltpu.async_copy(
                    nf_hbm.at[idx_v.at[g * G + b]],
                    rows_v.at[pl.ds(b * CHUNK, CHUNK)],
                    sem,
                )
                for b in range(G)
            ]
            for d in descs:
                d.wait()
            pltpu.sync_copy(
                rows_v, out_hbm.at[pl.ds(wid * EPT + g * (G * CHUNK), G * CHUNK)]
            )
            return carry

        lax.fori_loop(0, GITERS, body, 0)

    return _sc_gather


_BLK = 4096


def _tc_compute_body(eft_ref, f_ref, w1_ref, w2_ref, b2_ref, w3_ref, out_ref):
    h1 = jnp.maximum(
        lax.dot_general(
            eft_ref[...], w1_ref[...], (((0,), (0,)), ((), ())),
            preferred_element_type=jnp.float32,
        ),
        0.0,
    )
    h2 = jnp.maximum(
        jnp.dot(h1, w2_ref[...], preferred_element_type=jnp.float32)
        + b2_ref[0:1, :],
        0.0,
    )
    rw = jnp.dot(h2, w3_ref[...], preferred_element_type=jnp.float32)
    f = f_ref[...]
    ri = lax.broadcasted_iota(jnp.int32, (CIN, COUT * CIN), 0)
    rk = lax.broadcasted_iota(jnp.int32, (CIN, COUT * CIN), 1)
    rep = jnp.where((rk & 15) == ri, 1.0, 0.0)
    frep = jnp.dot(f, rep, preferred_element_type=jnp.float32)
    prod = rw * frep
    r = lax.broadcasted_iota(jnp.int32, (COUT * CIN, COUT), 0)
    c = lax.broadcasted_iota(jnp.int32, (COUT * CIN, COUT), 1)
    sel = jnp.where((r // CIN) == c, 1.0, 0.0)
    out_ref[...] = jnp.dot(prod, sel, preferred_element_type=jnp.float32)


def _tc_compute(eft, feats, w18, w2, b2_8, w3):
    grid = E_PAD // _BLK
    return pl.pallas_call(
        _tc_compute_body,
        grid=(grid,),
        in_specs=[
            pl.BlockSpec((8, _BLK), lambda i: (0, i)),
            pl.BlockSpec((_BLK, CIN), lambda i: (i, 0)),
            pl.BlockSpec((8, MID), lambda i: (0, 0)),
            pl.BlockSpec((MID, MID), lambda i: (0, 0)),
            pl.BlockSpec((8, MID), lambda i: (0, 0)),
            pl.BlockSpec((MID, COUT * CIN), lambda i: (0, 0)),
        ],
        out_specs=pl.BlockSpec((_BLK, COUT), lambda i: (i, 0)),
        out_shape=jax.ShapeDtypeStruct((E_PAD, COUT), jnp.float32),
    )(eft, feats, w18, w2, b2_8, w3)


@functools.cache
def _sc_scatter_kernel():
    @functools.partial(
        pl.kernel,
        out_type=jax.ShapeDtypeStruct((NC, NACC, COUT), jnp.float32),
        mesh=_mesh(),
        scratch_types=[
            pltpu.VMEM((CPT, CHUNK), jnp.int32),
            pltpu.VMEM((G * CHUNK, COUT), jnp.float32),
            pltpu.VMEM_SHARED((NACC, COUT), jnp.float32),
        ],
        compiler_params=pltpu.CompilerParams(use_tc_tiling_on_sc=False),
    )
    def _sc_scatter(tmp_hbm, dst_hbm, zero_hbm, out_hbm, idx_v, rows_v, acc):
        cid = lax.axis_index("c")
        sid = lax.axis_index("s")
        wid = sid * NC + cid
        pltpu.sync_copy(dst_hbm.at[wid], idx_v)

        @pl.when(sid == 0)
        def _():
            pltpu.sync_copy(zero_hbm, acc)

        plsc.subcore_barrier()

        def body(g, carry):
            pltpu.sync_copy(
                tmp_hbm.at[pl.ds(wid * EPT + g * (G * CHUNK), G * CHUNK)], rows_v
            )
            for b in range(G):
                pltpu.sync_copy(
                    rows_v.at[pl.ds(b * CHUNK, CHUNK)],
                    acc.at[idx_v.at[g * G + b]],
                    add=True,
                )
            return carry

        lax.fori_loop(0, GITERS, body, 0)
        plsc.subcore_barrier()
        pltpu.sync_copy(
            acc.at[pl.ds(sid * RPT, RPT)],
            out_hbm.at[cid, pl.ds(sid * RPT, RPT)],
        )

    return _sc_scatter


_ABLK = 2000


def _tc_add_body(a_ref, b_ref, out_ref):
    out_ref[...] = a_ref[...] + b_ref[...]


def _tc_add(a, b):
    grid = N // _ABLK
    return pl.pallas_call(
        _tc_add_body,
        grid=(grid,),
        in_specs=[
            pl.BlockSpec((_ABLK, COUT), lambda i: (i, 0)),
            pl.BlockSpec((_ABLK, COUT), lambda i: (i, 0)),
        ],
        out_specs=pl.BlockSpec((_ABLK, COUT), lambda i: (i, 0)),
        out_shape=jax.ShapeDtypeStruct((N, COUT), jnp.float32),
    )(a, b)


def kernel(node_feats_0, invariant_edge_feats, edge_index, W1, b1, W2, b2, W3):
    nf = node_feats_0.reshape(N, CIN)
    src = edge_index[0]
    dst = edge_index[1]
    pad = E_PAD - E

    src_p = jnp.concatenate([src, jnp.zeros((pad,), jnp.int32)]).reshape(NW, CPT, CHUNK)
    dst_p = jnp.concatenate([dst, jnp.full((pad,), N, jnp.int32)]).reshape(NW, CPT, CHUNK)

    eft = jnp.concatenate(
        [
            invariant_edge_feats.T,
            jnp.ones((1, E), jnp.float32),
            jnp.zeros((3, E), jnp.float32),
        ],
        axis=0,
    )
    eft = jnp.concatenate([eft, jnp.zeros((8, pad), jnp.float32)], axis=1)

    w18 = jnp.concatenate([W1, b1[None, :], jnp.zeros((3, MID), jnp.float32)], axis=0)
    b2_8 = jnp.broadcast_to(b2[None, :], (8, MID))

    feats = _sc_gather_kernel()(nf, src_p)
    tmp = _tc_compute(eft, feats, w18, W2, b2_8, W3)
    zero = jnp.zeros((NACC, COUT), jnp.float32)
    partials = _sc_scatter_kernel()(tmp, dst_p, zero)
    out = _tc_add(partials[0, :N], partials[1, :N])
    return out.reshape(N, CIN, 1)


# speedup vs baseline: 7.4522x; 1.5298x over previous
import functools

import jax
import jax.numpy as jnp
from jax import lax
from jax.experimental import pallas as pl
from jax.experimental.pallas import tpu as pltpu
from jax.experimental.pallas import tpu_sc as plsc

N = 50000
E = 800000
CIN = 16
COUT = 16
MID = 32

NC = 2
NS = 16
NW = NC * NS

CHUNK = 128
G = 14
GITERS = 14
CPT = G * GITERS
EPT = CPT * CHUNK
E_PAD = NW * EPT

NACC = 50048
RPT = NACC // NS

@functools.cache
def _mesh():
    return plsc.VectorSubcoreMesh(
        core_axis_name="c", subcore_axis_name="s", num_cores=NC, num_subcores=NS
    )


@functools.cache
def _sc_gather_kernel():
    @functools.partial(
        pl.kernel,
        out_type=jax.ShapeDtypeStruct((E_PAD, CIN), jnp.float32),
        mesh=_mesh(),
        scratch_types=[
            pltpu.VMEM((CPT, CHUNK), jnp.int32),
            pltpu.VMEM((G * CHUNK, CIN), jnp.float32),
            pltpu.SemaphoreType.DMA,
        ],
        compiler_params=pltpu.CompilerParams(use_tc_tiling_on_sc=False),
    )
    def _sc_gather(nf_hbm, src_hbm, out_hbm, idx_v, rows_v, sem):
        wid = lax.axis_index("s") * NC + lax.axis_index("c")
        pltpu.sync_copy(src_hbm.at[wid], idx_v)

        def body(g, carry):
            descs = [
                pltpu.async_copy(
                    nf_hbm.at[idx_v.at[g * G + b]],
                    rows_v.at[pl.ds(b * CHUNK, CHUNK)],
                    sem,
                )
                for b in range(G)
            ]
            for d in descs:
                d.wait()
            pltpu.sync_copy(
                rows_v, out_hbm.at[pl.ds(wid * EPT + g * (G * CHUNK), G * CHUNK)]
            )
            return carry

        lax.fori_loop(0, GITERS, body, 0)

    return _sc_gather


_BLK = 4096


def _tc_compute_body(eft_ref, f_ref, w1_ref, w2_ref, b2_ref, w3_ref, out_ref):
    h1 = jnp.maximum(
        lax.dot_general(
            eft_ref[...], w1_ref[...], (((0,), (0,)), ((), ())),
            preferred_element_type=jnp.float32,
        ),
        0.0,
    )
    h2 = jnp.maximum(
        jnp.dot(h1, w2_ref[...], preferred_element_type=jnp.float32)
        + b2_ref[0:1, :],
        0.0,
    )
    rw = jnp.dot(h2, w3_ref[...], preferred_element_type=jnp.float32)

    fp = f_ref[...]
    f8 = jnp.broadcast_to(fp[:, None, :], (_BLK // 8, 8, 128))
    f8 = jnp.reshape(f8, (_BLK, 128))
    me = lax.broadcasted_iota(jnp.int32, (_BLK, 128), 0)
    ml = lax.broadcasted_iota(jnp.int32, (_BLK, 128), 1)
    msk = jnp.where((ml >> 4) == (me & 7), 1.0, 0.0)
    fm = f8 * msk
    ci = lax.broadcasted_iota(jnp.int32, (128, CIN), 0)
    cj = lax.broadcasted_iota(jnp.int32, (128, CIN), 1)
    c2 = jnp.where((ci & 15) == cj, 1.0, 0.0)
    f = jnp.dot(fm, c2, preferred_element_type=jnp.float32)

    ri = lax.broadcasted_iota(jnp.int32, (CIN, COUT * CIN), 0)
    rk = lax.broadcasted_iota(jnp.int32, (CIN, COUT * CIN), 1)
    rep = jnp.where((rk & 15) == ri, 1.0, 0.0)
    frep = jnp.dot(f, rep, preferred_element_type=jnp.float32)
    prod = rw * frep
    r = lax.broadcasted_iota(jnp.int32, (COUT * CIN, COUT), 0)
    c = lax.broadcasted_iota(jnp.int32, (COUT * CIN, COUT), 1)
    sel = jnp.where((r // CIN) == c, 1.0, 0.0)
    tmp = jnp.dot(prod, sel, preferred_element_type=jnp.float32)

    di = lax.broadcasted_iota(jnp.int32, (COUT, 128), 0)
    dl = lax.broadcasted_iota(jnp.int32, (COUT, 128), 1)
    dmat = jnp.where((dl & 15) == di, 1.0, 0.0)
    t128 = jnp.dot(tmp, dmat, preferred_element_type=jnp.float32)
    tm = t128 * msk
    out_ref[...] = jnp.sum(jnp.reshape(tm, (_BLK // 8, 8, 128)), axis=1)


def _tc_compute(eft, feats_p, w18, w2, b2_8, w3):
    grid = E_PAD // _BLK
    return pl.pallas_call(
        _tc_compute_body,
        grid=(grid,),
        in_specs=[
            pl.BlockSpec((8, _BLK), lambda i: (0, i)),
            pl.BlockSpec((_BLK // 8, 128), lambda i: (i, 0)),
            pl.BlockSpec((8, MID), lambda i: (0, 0)),
            pl.BlockSpec((MID, MID), lambda i: (0, 0)),
            pl.BlockSpec((8, MID), lambda i: (0, 0)),
            pl.BlockSpec((MID, COUT * CIN), lambda i: (0, 0)),
        ],
        out_specs=pl.BlockSpec((_BLK // 8, 128), lambda i: (i, 0)),
        out_shape=jax.ShapeDtypeStruct((E_PAD // 8, 128), jnp.float32),
    )(eft, feats_p, w18, w2, b2_8, w3)


@functools.cache
def _sc_scatter_kernel():
    @functools.partial(
        pl.kernel,
        out_type=jax.ShapeDtypeStruct((NC, NACC, COUT), jnp.float32),
        mesh=_mesh(),
        scratch_types=[
            pltpu.VMEM((CPT, CHUNK), jnp.int32),
            pltpu.VMEM((G * CHUNK, COUT), jnp.float32),
            pltpu.VMEM_SHARED((NACC, COUT), jnp.float32),
        ],
        compiler_params=pltpu.CompilerParams(use_tc_tiling_on_sc=False),
    )
    def _sc_scatter(tmp_hbm, dst_hbm, zero_hbm, out_hbm, idx_v, rows_v, acc):
        cid = lax.axis_index("c")
        sid = lax.axis_index("s")
        wid = sid * NC + cid
        pltpu.sync_copy(dst_hbm.at[wid], idx_v)

        @pl.when(sid == 0)
        def _():
            pltpu.sync_copy(zero_hbm, acc)

        plsc.subcore_barrier()

        def body(g, carry):
            pltpu.sync_copy(
                tmp_hbm.at[pl.ds(wid * EPT + g * (G * CHUNK), G * CHUNK)], rows_v
            )
            for b in range(G):
                pltpu.sync_copy(
                    rows_v.at[pl.ds(b * CHUNK, CHUNK)],
                    acc.at[idx_v.at[g * G + b]],
                    add=True,
                )
            return carry

        lax.fori_loop(0, GITERS, body, 0)
        plsc.subcore_barrier()
        pltpu.sync_copy(
            acc.at[pl.ds(sid * RPT, RPT)],
            out_hbm.at[cid, pl.ds(sid * RPT, RPT)],
        )

    return _sc_scatter


_APACK = NACC * COUT // 128


def _tc_add(ab):
    grid = 2
    blk = _APACK // grid
    return pl.pallas_call(
        lambda a_ref, out_ref: out_ref.__setitem__(
            (slice(None), slice(None)), a_ref[0] + a_ref[1]
        ),
        grid=(grid,),
        in_specs=[pl.BlockSpec((2, blk, 128), lambda i: (0, i, 0))],
        out_specs=pl.BlockSpec((blk, 128), lambda i: (i, 0)),
        out_shape=jax.ShapeDtypeStruct((_APACK, 128), jnp.float32),
    )(ab)


def kernel(node_feats_0, invariant_edge_feats, edge_index, W1, b1, W2, b2, W3):
    nf = node_feats_0.reshape(N, CIN)
    src = edge_index[0]
    dst = edge_index[1]
    pad = E_PAD - E

    src_p = jnp.concatenate([src, jnp.zeros((pad,), jnp.int32)]).reshape(NW, CPT, CHUNK)
    dst_p = jnp.concatenate([dst, jnp.full((pad,), N, jnp.int32)]).reshape(NW, CPT, CHUNK)

    eft = jnp.concatenate(
        [
            invariant_edge_feats.T,
            jnp.ones((1, E), jnp.float32),
            jnp.zeros((3, E), jnp.float32),
        ],
        axis=0,
    )
    eft = jnp.concatenate([eft, jnp.zeros((8, pad), jnp.float32)], axis=1)

    w18 = jnp.concatenate([W1, b1[None, :], jnp.zeros((3, MID), jnp.float32)], axis=0)
    b2_8 = jnp.broadcast_to(b2[None, :], (8, MID))

    feats = _sc_gather_kernel()(nf, src_p)
    feats_p = feats.reshape(E_PAD // 8, 128)
    tmp_p = _tc_compute(eft, feats_p, w18, W2, b2_8, W3)
    tmp = tmp_p.reshape(E_PAD, COUT)
    zero = jnp.zeros((NACC, COUT), jnp.float32)
    partials = _sc_scatter_kernel()(tmp, dst_p, zero)
    out_p = _tc_add(partials.reshape(NC, _APACK, 128))
    out = out_p.reshape(NACC, COUT)[:N]
    return out.reshape(N, CIN, 1)


# speedup vs baseline: 8.6890x; 1.1660x over previous
import functools

import jax
import jax.numpy as jnp
from jax import lax
from jax.experimental import pallas as pl
from jax.experimental.pallas import tpu as pltpu
from jax.experimental.pallas import tpu_sc as plsc

N = 50000
E = 800000
CIN = 16
COUT = 16
MID = 32

NC = 2
NS = 16
NW = NC * NS

CHUNK = 128
G = 14
GITERS = 14
CPT = G * GITERS
EPT = CPT * CHUNK
E_PAD = NW * EPT

NACC = 50048
RPT = NACC // NS

@functools.cache
def _mesh():
    return plsc.VectorSubcoreMesh(
        core_axis_name="c", subcore_axis_name="s", num_cores=NC, num_subcores=NS
    )


@functools.cache
def _sc_gather_kernel():
    @functools.partial(
        pl.kernel,
        out_type=jax.ShapeDtypeStruct((E_PAD, CIN), jnp.float32),
        mesh=_mesh(),
        scratch_types=[
            pltpu.VMEM((CPT, CHUNK), jnp.int32),
            pltpu.VMEM((G * CHUNK, CIN), jnp.float32),
            pltpu.SemaphoreType.DMA,
        ],
        compiler_params=pltpu.CompilerParams(use_tc_tiling_on_sc=False),
    )
    def _sc_gather(nf_hbm, src_hbm, out_hbm, idx_v, rows_v, sem):
        wid = lax.axis_index("s") * NC + lax.axis_index("c")
        pltpu.sync_copy(src_hbm.at[wid], idx_v)

        def body(g, carry):
            descs = [
                pltpu.async_copy(
                    nf_hbm.at[idx_v.at[g * G + b]],
                    rows_v.at[pl.ds(b * CHUNK, CHUNK)],
                    sem,
                )
                for b in range(G)
            ]
            for d in descs:
                d.wait()
            pltpu.sync_copy(
                rows_v, out_hbm.at[pl.ds(wid * EPT + g * (G * CHUNK), G * CHUNK)]
            )
            return carry

        lax.fori_loop(0, GITERS, body, 0)

    return _sc_gather


_BLK = 4096


def _tc_compute_body(eft_ref, f_ref, w1_ref, w2_ref, b2c_ref, w3_ref, out_ref):
    eft = eft_ref[...]
    h1t = jnp.maximum(
        lax.dot_general(w1_ref[...], eft, (((0,), (0,)), ((), ())),
                        preferred_element_type=jnp.float32),
        0.0,
    )
    h2t = jnp.maximum(
        lax.dot_general(w2_ref[...], h1t, (((0,), (0,)), ((), ())),
                        preferred_element_type=jnp.float32)
        + b2c_ref[:, 0:1],
        0.0,
    )
    rw = lax.dot_general(h2t, w3_ref[...], (((0,), (0,)), ((), ())),
                         preferred_element_type=jnp.float32)

    fp = f_ref[...]
    f8 = jnp.broadcast_to(fp[:, None, :], (_BLK // 8, 8, 128))
    f8 = jnp.reshape(f8, (_BLK, 128))
    me = lax.broadcasted_iota(jnp.int32, (_BLK, 128), 0)
    ml = lax.broadcasted_iota(jnp.int32, (_BLK, 128), 1)
    msk = jnp.where((ml >> 4) == (me & 7), 1.0, 0.0)
    fm = f8 * msk
    ci = lax.broadcasted_iota(jnp.int32, (128, COUT * CIN), 0)
    cj = lax.broadcasted_iota(jnp.int32, (128, COUT * CIN), 1)
    cmat = jnp.where((ci & 15) == (cj & 15), 1.0, 0.0)
    frep = jnp.dot(fm, cmat, preferred_element_type=jnp.float32)

    prod = rw * frep
    si = lax.broadcasted_iota(jnp.int32, (COUT * CIN, 128), 0)
    sc = lax.broadcasted_iota(jnp.int32, (COUT * CIN, 128), 1)
    sd = jnp.where((sc & 15) == (si >> 4), 1.0, 0.0)
    t128 = jnp.dot(prod, sd, preferred_element_type=jnp.float32)
    tm = t128 * msk
    out_ref[...] = jnp.sum(jnp.reshape(tm, (_BLK // 8, 8, 128)), axis=1)


def _tc_compute(eft, feats_p, w18, w2, b2c, w3):
    grid = E_PAD // _BLK
    return pl.pallas_call(
        _tc_compute_body,
        grid=(grid,),
        in_specs=[
            pl.BlockSpec((8, _BLK), lambda i: (0, i)),
            pl.BlockSpec((_BLK // 8, 128), lambda i: (i, 0)),
            pl.BlockSpec((8, MID), lambda i: (0, 0)),
            pl.BlockSpec((MID, MID), lambda i: (0, 0)),
            pl.BlockSpec((MID, 128), lambda i: (0, 0)),
            pl.BlockSpec((MID, COUT * CIN), lambda i: (0, 0)),
        ],
        out_specs=pl.BlockSpec((_BLK // 8, 128), lambda i: (i, 0)),
        out_shape=jax.ShapeDtypeStruct((E_PAD // 8, 128), jnp.float32),
    )(eft, feats_p, w18, w2, b2c, w3)


@functools.cache
def _sc_scatter_kernel():
    @functools.partial(
        pl.kernel,
        out_type=jax.ShapeDtypeStruct((NC, NACC, COUT), jnp.float32),
        mesh=_mesh(),
        scratch_types=[
            pltpu.VMEM((CPT, CHUNK), jnp.int32),
            pltpu.VMEM((G * CHUNK, COUT), jnp.float32),
            pltpu.VMEM_SHARED((NACC, COUT), jnp.float32),
        ],
        compiler_params=pltpu.CompilerParams(use_tc_tiling_on_sc=False),
    )
    def _sc_scatter(tmp_hbm, dst_hbm, zero_hbm, out_hbm, idx_v, rows_v, acc):
        cid = lax.axis_index("c")
        sid = lax.axis_index("s")
        wid = sid * NC + cid
        pltpu.sync_copy(dst_hbm.at[wid], idx_v)

        @pl.when(sid == 0)
        def _():
            pltpu.sync_copy(zero_hbm, acc)

        plsc.subcore_barrier()

        def body(g, carry):
            pltpu.sync_copy(
                tmp_hbm.at[pl.ds(wid * EPT + g * (G * CHUNK), G * CHUNK)], rows_v
            )
            for b in range(G):
                pltpu.sync_copy(
                    rows_v.at[pl.ds(b * CHUNK, CHUNK)],
                    acc.at[idx_v.at[g * G + b]],
                    add=True,
                )
            return carry

        lax.fori_loop(0, GITERS, body, 0)
        plsc.subcore_barrier()
        pltpu.sync_copy(
            acc.at[pl.ds(sid * RPT, RPT)],
            out_hbm.at[cid, pl.ds(sid * RPT, RPT)],
        )

    return _sc_scatter


_APACK = NACC * COUT // 128


def _tc_add(ab):
    grid = 2
    blk = _APACK // grid
    return pl.pallas_call(
        lambda a_ref, out_ref: out_ref.__setitem__(
            (slice(None), slice(None)), a_ref[0] + a_ref[1]
        ),
        grid=(grid,),
        in_specs=[pl.BlockSpec((2, blk, 128), lambda i: (0, i, 0))],
        out_specs=pl.BlockSpec((blk, 128), lambda i: (i, 0)),
        out_shape=jax.ShapeDtypeStruct((_APACK, 128), jnp.float32),
    )(ab)


def kernel(node_feats_0, invariant_edge_feats, edge_index, W1, b1, W2, b2, W3):
    nf = node_feats_0.reshape(N, CIN)
    src = edge_index[0]
    dst = edge_index[1]
    pad = E_PAD - E

    src_p = jnp.concatenate([src, jnp.zeros((pad,), jnp.int32)]).reshape(NW, CPT, CHUNK)
    dst_p = jnp.concatenate([dst, jnp.full((pad,), N, jnp.int32)]).reshape(NW, CPT, CHUNK)

    eft = jnp.concatenate(
        [
            invariant_edge_feats.T,
            jnp.ones((1, E), jnp.float32),
            jnp.zeros((3, E), jnp.float32),
        ],
        axis=0,
    )
    eft = jnp.concatenate([eft, jnp.zeros((8, pad), jnp.float32)], axis=1)

    w18 = jnp.concatenate([W1, b1[None, :], jnp.zeros((3, MID), jnp.float32)], axis=0)
    b2c = jnp.broadcast_to(b2[:, None], (MID, 128))

    feats = _sc_gather_kernel()(nf, src_p)
    feats_p = feats.reshape(E_PAD // 8, 128)
    tmp_p = _tc_compute(eft, feats_p, w18, W2, b2c, W3)
    tmp = tmp_p.reshape(E_PAD, COUT)
    zero = jnp.zeros((NACC, COUT), jnp.float32)
    partials = _sc_scatter_kernel()(tmp, dst_p, zero)
    out_p = _tc_add(partials.reshape(NC, _APACK, 128))
    out = out_p.reshape(NACC, COUT)[:N]
    return out.reshape(N, CIN, 1)
